# initial kernel scaffold (unmeasured)
import jax
import jax.numpy as jnp
from jax import lax
from jax.experimental import pallas as pl
from jax.experimental.pallas import tpu as pltpu


def kernel(
    x,
):
    def body(*refs):
        pass

    out_shape = jax.ShapeDtypeStruct(..., jnp.float32)
    return pl.pallas_call(body, out_shape=out_shape)(...)


# baseline (device time: 28004 ns/iter reference)
import jax
import jax.numpy as jnp
from jax import lax
from jax.experimental import pallas as pl
from jax.experimental.pallas import tpu as pltpu

N_DEV = 16
LOG2_N = 4


def kernel(x):
    _, m, n = x.shape

    def body(x_ref, out_ref, comm_ref, send_sems, recv_sems):
        my = lax.axis_index("i")

        barrier_sem = pltpu.get_barrier_semaphore()
        for k in range(LOG2_N):
            partner = my ^ (1 << k)
            pl.semaphore_signal(
                barrier_sem, inc=1,
                device_id=(partner,), device_id_type=pl.DeviceIdType.MESH,
            )
        pl.semaphore_wait(barrier_sem, LOG2_N)

        out_ref[...] = x_ref[0]

        for k in range(LOG2_N):
            partner = my ^ (1 << k)
            rdma = pltpu.make_async_remote_copy(
                src_ref=out_ref,
                dst_ref=comm_ref.at[k],
                send_sem=send_sems.at[k],
                recv_sem=recv_sems.at[k],
                device_id=(partner,),
                device_id_type=pl.DeviceIdType.MESH,
            )
            rdma.start()
            rdma.wait()
            out_ref[...] = out_ref[...] + comm_ref[k]

    return pl.pallas_call(
        body,
        out_shape=jax.ShapeDtypeStruct((m, n), x.dtype),
        in_specs=[pl.BlockSpec(memory_space=pltpu.VMEM)],
        out_specs=pl.BlockSpec(memory_space=pltpu.VMEM),
        scratch_shapes=[
            pltpu.VMEM((LOG2_N, m, n), x.dtype),
            pltpu.SemaphoreType.DMA((LOG2_N,)),
            pltpu.SemaphoreType.DMA((LOG2_N,)),
        ],
        compiler_params=pltpu.CompilerParams(collective_id=0),
    )(x)


# device time: 17803 ns/iter; 1.5730x vs baseline; 1.5730x over previous
import jax
import jax.numpy as jnp
from jax import lax
from jax.experimental import pallas as pl
from jax.experimental.pallas import tpu as pltpu

N_DEV = 16


def kernel(x):
    _, m, n = x.shape
    rows = m // N_DEV

    def body(x_ref, out_ref, rs_buf, rs_send, rs_recv, ag_send, ag_recv):
        my = lax.axis_index("i")

        barrier_sem = pltpu.get_barrier_semaphore()
        for r in range(1, N_DEV):
            pl.semaphore_signal(
                barrier_sem, inc=1,
                device_id=(lax.rem(my + r, N_DEV),),
                device_id_type=pl.DeviceIdType.MESH,
            )
        pl.semaphore_wait(barrier_sem, N_DEV - 1)

        rs_buf[0] = x_ref[pl.ds(my * rows, rows), :]

        for r in range(1, N_DEV):
            tgt = lax.rem(my + r, N_DEV)
            pltpu.make_async_remote_copy(
                src_ref=x_ref.at[pl.ds(tgt * rows, rows), :],
                dst_ref=rs_buf.at[N_DEV - r],
                send_sem=rs_send.at[r],
                recv_sem=rs_recv.at[N_DEV - r],
                device_id=(tgt,),
                device_id_type=pl.DeviceIdType.MESH,
            ).start()

        for s in range(1, N_DEV):
            pltpu.make_async_copy(rs_buf.at[s], rs_buf.at[s], rs_recv.at[s]).wait()
        out_ref[pl.ds(my * rows, rows), :] = jnp.sum(rs_buf[...], axis=0)

        for r in range(1, N_DEV):
            tgt = lax.rem(my + r, N_DEV)
            pltpu.make_async_remote_copy(
                src_ref=out_ref.at[pl.ds(my * rows, rows), :],
                dst_ref=out_ref.at[pl.ds(my * rows, rows), :],
                send_sem=ag_send.at[r],
                recv_sem=ag_recv.at[N_DEV - r],
                device_id=(tgt,),
                device_id_type=pl.DeviceIdType.MESH,
            ).start()

        for s in range(1, N_DEV):
            pltpu.make_async_copy(
                out_ref.at[pl.ds(0, rows), :],
                out_ref.at[pl.ds(0, rows), :],
                ag_recv.at[s],
            ).wait()
        for r in range(1, N_DEV):
            pltpu.make_async_copy(
                out_ref.at[pl.ds(0, rows), :],
                out_ref.at[pl.ds(0, rows), :],
                rs_send.at[r],
            ).wait()
            pltpu.make_async_copy(
                out_ref.at[pl.ds(0, rows), :],
                out_ref.at[pl.ds(0, rows), :],
                ag_send.at[r],
            ).wait()

    x2 = x.reshape(m, n)
    return pl.pallas_call(
        body,
        out_shape=jax.ShapeDtypeStruct((m, n), x.dtype),
        in_specs=[pl.BlockSpec(memory_space=pltpu.VMEM)],
        out_specs=pl.BlockSpec(memory_space=pltpu.VMEM),
        scratch_shapes=[
            pltpu.VMEM((N_DEV, rows, n), x.dtype),
            pltpu.SemaphoreType.DMA((N_DEV,)),
            pltpu.SemaphoreType.DMA((N_DEV,)),
            pltpu.SemaphoreType.DMA((N_DEV,)),
            pltpu.SemaphoreType.DMA((N_DEV,)),
        ],
        compiler_params=pltpu.CompilerParams(collective_id=0),
    )(x2)
